# Initial kernel scaffold; baseline (speedup 1.0000x reference)
import functools

import jax
import jax.numpy as jnp
from jax import lax
from jax.experimental import pallas as pl
from jax.experimental.pallas import tpu as pltpu
from jax.experimental.pallas import tpu_sc as plsc

ALPHA = 0.1
NITER = 10
NSC = 2
NTILE = 16
LANES = 16
CHUNK = 128
PAD_ROWS = 16


def _mlp_body(x_ref, w1_ref, w2_ref, o_ref):
    h = lax.dot_general(
        x_ref[...], w1_ref[...], (((1,), (0,)), ((), ())),
        precision=lax.Precision.HIGHEST, preferred_element_type=jnp.float32)
    h = jnp.maximum(h, 0.0)
    o_ref[...] = lax.dot_general(
        h, w2_ref[...], (((1,), (0,)), ((), ())),
        precision=lax.Precision.HIGHEST, preferred_element_type=jnp.float32)


def _mlp(x, W1, W2):
    n, d_feat = x.shape
    hidden = W1.shape[1]
    nclasses = W2.shape[1]
    rows = 1000
    grid = n // rows
    return pl.pallas_call(
        _mlp_body,
        grid=(grid,),
        in_specs=[
            pl.BlockSpec((rows, d_feat), lambda i: (i, 0)),
            pl.BlockSpec((d_feat, hidden), lambda i: (0, 0)),
            pl.BlockSpec((hidden, nclasses), lambda i: (0, 0)),
        ],
        out_specs=pl.BlockSpec((rows, nclasses), lambda i: (i, 0)),
        out_shape=jax.ShapeDtypeStruct((n, nclasses), jnp.float32),
    )(x, W1, W2)


def _rsqrt_newton(d):
    i = plsc.bitcast(d, jnp.int32)
    i = 0x5F3759DF - lax.shift_right_logical(i, 1)
    y = plsc.bitcast(i, jnp.float32)
    for _ in range(3):
        y = y * (1.5 - 0.5 * d * y * y)
    return y


def _propagate(h_hbm, src_hbm, dst_hbm, n, nchunk):
    rows_pt = n // NTILE
    nclasses = h_hbm.shape[1]
    tab_rows = n + PAD_ROWS

    mesh = plsc.VectorSubcoreMesh(
        core_axis_name="c", subcore_axis_name="s",
        num_cores=NSC, num_subcores=NTILE)

    @functools.partial(
        pl.kernel,
        mesh=mesh,
        out_type=jax.ShapeDtypeStruct((n, nclasses), jnp.float32),
        scratch_types=[
            pltpu.VMEM_SHARED((tab_rows, nclasses), jnp.float32),
            pltpu.VMEM_SHARED((tab_rows, nclasses), jnp.float32),
            pltpu.VMEM((nchunk, CHUNK), jnp.int32),
            pltpu.VMEM((nchunk, CHUNK), jnp.int32),
            pltpu.VMEM((CHUNK, nclasses), jnp.float32),
            pltpu.VMEM((CHUNK, nclasses), jnp.float32),
            pltpu.VMEM((n // NTILE, nclasses), jnp.float32),
            pltpu.VMEM((n // NTILE, nclasses), jnp.float32),
            pltpu.VMEM((n // NTILE, nclasses), jnp.float32),
            pltpu.VMEM((n // NTILE, nclasses), jnp.float32),
            pltpu.VMEM((n // NTILE, nclasses), jnp.float32),
            pltpu.SemaphoreType.DMA,
        ],
    )
    def k(h_r, src_r, dst_r, out_r,
          y_tab, s_tab, src_v, dst_v, rows_v, ones_v, zeros_v,
          h_v, dx_v, sb_v, yb_v, sem):
        wid = lax.axis_index("s")
        cid = lax.axis_index("c")
        r0 = wid * rows_pt

        pltpu.sync_copy(src_r.at[wid], src_v)
        pltpu.sync_copy(dst_r.at[wid], dst_v)
        pltpu.sync_copy(h_r.at[pl.ds(r0, rows_pt)], h_v)

        def fill_ones(i, _):
            ones_v[i, :] = jnp.full((LANES,), 1.0, jnp.float32)
            return 0
        lax.fori_loop(0, CHUNK, fill_ones, 0)

        def fill_zeros(i, _):
            zeros_v[i, :] = jnp.zeros((LANES,), jnp.float32)
            return 0
        lax.fori_loop(0, rows_pt, fill_zeros, 0)

        pltpu.sync_copy(zeros_v, s_tab.at[pl.ds(r0, rows_pt)])
        plsc.subcore_barrier()

        def deg_step(c, _):
            pltpu.sync_copy(ones_v, s_tab.at[dst_v.at[c]], add=True)
            return 0
        lax.fori_loop(0, nchunk, deg_step, 0)
        plsc.subcore_barrier()

        pltpu.sync_copy(s_tab.at[pl.ds(r0, rows_pt)], sb_v)
        pltpu.sync_copy(zeros_v, s_tab.at[pl.ds(r0, rows_pt)])

        def init_step(f, _):
            deg = sb_v[f, :] + 1.0
            dx = _rsqrt_newton(deg)
            dx_v[f, :] = dx
            sb_v[f, :] = dx * h_v[f, :]
            return 0
        lax.fori_loop(0, rows_pt, init_step, 0)
        pltpu.sync_copy(sb_v, y_tab.at[pl.ds(r0, rows_pt)])
        plsc.subcore_barrier()

        for _ in range(NITER):
            def edge_step(c, _):
                pltpu.async_copy(y_tab.at[src_v.at[c]], rows_v, sem).wait()
                pltpu.sync_copy(rows_v, s_tab.at[dst_v.at[c]], add=True)
                return 0
            lax.fori_loop(0, nchunk, edge_step, 0)
            plsc.subcore_barrier()

            pltpu.sync_copy(s_tab.at[pl.ds(r0, rows_pt)], sb_v)
            pltpu.sync_copy(y_tab.at[pl.ds(r0, rows_pt)], yb_v)
            pltpu.sync_copy(zeros_v, s_tab.at[pl.ds(r0, rows_pt)])

            def upd_step(f, _):
                dx = dx_v[f, :]
                v = sb_v[f, :] + yb_v[f, :]
                sb_v[f, :] = ((1.0 - ALPHA) * dx * dx) * v + (ALPHA * dx) * h_v[f, :]
                return 0
            lax.fori_loop(0, rows_pt, upd_step, 0)
            pltpu.sync_copy(sb_v, y_tab.at[pl.ds(r0, rows_pt)])
            plsc.subcore_barrier()

        def out_step(f, _):
            yb_v[f, :] = sb_v[f, :] / dx_v[f, :]
            return 0
        lax.fori_loop(0, rows_pt, out_step, 0)

        @pl.when(cid == 0)
        def _():
            pltpu.sync_copy(yb_v, out_r.at[pl.ds(r0, rows_pt)])

    return k(h_hbm, src_hbm, dst_hbm)


def kernel(x, edge_index, W1, W2):
    n = x.shape[0]
    e = edge_index.shape[1]
    assert n % NTILE == 0

    h = _mlp(x, W1, W2)

    per_tile = -(-e // (NTILE * CHUNK)) * CHUNK
    e_pad = per_tile * NTILE
    pad = jnp.arange(e_pad - e, dtype=jnp.int32) % PAD_ROWS + n
    src = jnp.concatenate([edge_index[0], pad]).reshape(NTILE, per_tile // CHUNK, CHUNK)
    dst = jnp.concatenate([edge_index[1], pad]).reshape(NTILE, per_tile // CHUNK, CHUNK)

    return _propagate(h, src, dst, n, per_tile // CHUNK)


# speedup vs baseline: 36.2571x; 36.2571x over previous
import functools

import jax
import jax.numpy as jnp
from jax import lax
from jax.experimental import pallas as pl
from jax.experimental.pallas import tpu as pltpu
from jax.experimental.pallas import tpu_sc as plsc

ALPHA = 0.1
NITER = 10
NSC = 2
NTILE = 16
LANES = 16
CHUNK = 128
PAD_ROWS = 16


def _mlp_body(x_ref, w1_ref, w2_ref, o_ref):
    h = lax.dot_general(
        x_ref[...], w1_ref[...], (((1,), (0,)), ((), ())),
        precision=lax.Precision.HIGHEST, preferred_element_type=jnp.float32)
    h = jnp.maximum(h, 0.0)
    o_ref[...] = lax.dot_general(
        h, w2_ref[...], (((1,), (0,)), ((), ())),
        precision=lax.Precision.HIGHEST, preferred_element_type=jnp.float32)


def _mlp(x, W1, W2):
    n, d_feat = x.shape
    hidden = W1.shape[1]
    nclasses = W2.shape[1]
    rows = 1000
    grid = n // rows
    return pl.pallas_call(
        _mlp_body,
        grid=(grid,),
        in_specs=[
            pl.BlockSpec((rows, d_feat), lambda i: (i, 0)),
            pl.BlockSpec((d_feat, hidden), lambda i: (0, 0)),
            pl.BlockSpec((hidden, nclasses), lambda i: (0, 0)),
        ],
        out_specs=pl.BlockSpec((rows, nclasses), lambda i: (i, 0)),
        out_shape=jax.ShapeDtypeStruct((n, nclasses), jnp.float32),
    )(x, W1, W2)


def _rsqrt_newton(d):
    i = plsc.bitcast(d, jnp.int32)
    i = 0x5F3759DF - lax.shift_right_logical(i, 1)
    y = plsc.bitcast(i, jnp.float32)
    for _ in range(3):
        y = y * (1.5 - 0.5 * d * y * y)
    return y


def _propagate(h_hbm, src_hbm, dst_hbm, n, nchunk):
    rows_pt = n // NTILE
    nclasses = h_hbm.shape[1]
    tab_rows = n + PAD_ROWS

    mesh = plsc.VectorSubcoreMesh(
        core_axis_name="c", subcore_axis_name="s",
        num_cores=NSC, num_subcores=NTILE)

    @functools.partial(
        pl.kernel,
        mesh=mesh,
        compiler_params=pltpu.CompilerParams(
            needs_layout_passes=False, use_tc_tiling_on_sc=False),
        out_type=jax.ShapeDtypeStruct((NTILE, rows_pt, nclasses), jnp.float32),
        scratch_types=[
            pltpu.VMEM_SHARED((tab_rows, nclasses), jnp.float32),
            pltpu.VMEM_SHARED((tab_rows, nclasses), jnp.float32),
            pltpu.VMEM((nchunk, CHUNK), jnp.int32),
            pltpu.VMEM((nchunk, CHUNK), jnp.int32),
            pltpu.VMEM((CHUNK, nclasses), jnp.float32),
            pltpu.VMEM((CHUNK, nclasses), jnp.float32),
            pltpu.VMEM((n // NTILE, nclasses), jnp.float32),
            pltpu.VMEM((n // NTILE, nclasses), jnp.float32),
            pltpu.VMEM((n // NTILE, nclasses), jnp.float32),
            pltpu.VMEM((n // NTILE, nclasses), jnp.float32),
            pltpu.VMEM((n // NTILE, nclasses), jnp.float32),
            pltpu.SemaphoreType.DMA,
        ],
    )
    def k(h_r, src_r, dst_r, out_r,
          y_tab, s_tab, src_v, dst_v, rows_v, ones_v, zeros_v,
          h_v, dx_v, sb_v, yb_v, sem):
        wid = lax.axis_index("s")
        cid = lax.axis_index("c")
        r0 = wid * rows_pt

        pltpu.sync_copy(src_r.at[wid], src_v)
        pltpu.sync_copy(dst_r.at[wid], dst_v)
        pltpu.sync_copy(h_r.at[wid], h_v)

        def fill_ones(i, _):
            ones_v[i, :] = jnp.full((LANES,), 1.0, jnp.float32)
            return 0
        lax.fori_loop(0, CHUNK, fill_ones, 0)

        def fill_zeros(i, _):
            zeros_v[i, :] = jnp.zeros((LANES,), jnp.float32)
            return 0
        lax.fori_loop(0, rows_pt, fill_zeros, 0)

        pltpu.sync_copy(zeros_v, s_tab.at[pl.ds(r0, rows_pt)])
        plsc.subcore_barrier()

        def deg_step(c, _):
            pltpu.sync_copy(ones_v, s_tab.at[dst_v.at[c]], add=True)
            return 0
        lax.fori_loop(0, nchunk, deg_step, 0)
        plsc.subcore_barrier()

        pltpu.sync_copy(s_tab.at[pl.ds(r0, rows_pt)], sb_v)
        pltpu.sync_copy(zeros_v, s_tab.at[pl.ds(r0, rows_pt)])

        def init_step(f, _):
            deg = sb_v[f, :] + 1.0
            dx = _rsqrt_newton(deg)
            dx_v[f, :] = dx
            sb_v[f, :] = dx * h_v[f, :]
            return 0
        lax.fori_loop(0, rows_pt, init_step, 0)
        pltpu.sync_copy(sb_v, y_tab.at[pl.ds(r0, rows_pt)])
        plsc.subcore_barrier()

        for _ in range(NITER):
            def edge_step(c, _):
                pltpu.async_copy(y_tab.at[src_v.at[c]], rows_v, sem).wait()
                pltpu.sync_copy(rows_v, s_tab.at[dst_v.at[c]], add=True)
                return 0
            lax.fori_loop(0, nchunk, edge_step, 0)
            plsc.subcore_barrier()

            pltpu.sync_copy(s_tab.at[pl.ds(r0, rows_pt)], sb_v)
            pltpu.sync_copy(y_tab.at[pl.ds(r0, rows_pt)], yb_v)
            pltpu.sync_copy(zeros_v, s_tab.at[pl.ds(r0, rows_pt)])

            def upd_step(f, _):
                dx = dx_v[f, :]
                v = sb_v[f, :] + yb_v[f, :]
                sb_v[f, :] = ((1.0 - ALPHA) * dx * dx) * v + (ALPHA * dx) * h_v[f, :]
                return 0
            lax.fori_loop(0, rows_pt, upd_step, 0)
            pltpu.sync_copy(sb_v, y_tab.at[pl.ds(r0, rows_pt)])
            plsc.subcore_barrier()

        def out_step(f, _):
            yb_v[f, :] = sb_v[f, :] / dx_v[f, :]
            return 0
        lax.fori_loop(0, rows_pt, out_step, 0)

        @pl.when(cid == 0)
        def _():
            pltpu.sync_copy(yb_v, out_r.at[wid])

    out3 = k(h_hbm.reshape(NTILE, rows_pt, nclasses), src_hbm, dst_hbm)
    return out3.reshape(n, nclasses)


def kernel(x, edge_index, W1, W2):
    n = x.shape[0]
    e = edge_index.shape[1]
    assert n % NTILE == 0

    h = _mlp(x, W1, W2)

    per_tile = -(-e // (NTILE * CHUNK)) * CHUNK
    e_pad = per_tile * NTILE
    pad = jnp.arange(e_pad - e, dtype=jnp.int32) % PAD_ROWS + n
    src = jnp.concatenate([edge_index[0], pad]).reshape(NTILE, per_tile // CHUNK, CHUNK)
    dst = jnp.concatenate([edge_index[1], pad]).reshape(NTILE, per_tile // CHUNK, CHUNK)

    return _propagate(h, src, dst, n, per_tile // CHUNK)


# speedup vs baseline: 48.7453x; 1.3444x over previous
import functools

import jax
import jax.numpy as jnp
from jax import lax
from jax.experimental import pallas as pl
from jax.experimental.pallas import tpu as pltpu
from jax.experimental.pallas import tpu_sc as plsc

ALPHA = 0.1
NITER = 10
NSC = 2
NTILE = 16
LANES = 16
CHUNK = 128
PAD_ROWS = 16


def _mlp_body(x_ref, w1_ref, w2_ref, o_ref):
    h = lax.dot_general(
        x_ref[...], w1_ref[...], (((1,), (0,)), ((), ())),
        precision=lax.Precision.HIGHEST, preferred_element_type=jnp.float32)
    h = jnp.maximum(h, 0.0)
    o_ref[...] = lax.dot_general(
        h, w2_ref[...], (((1,), (0,)), ((), ())),
        precision=lax.Precision.HIGHEST, preferred_element_type=jnp.float32)


def _mlp(x, W1, W2):
    n, d_feat = x.shape
    hidden = W1.shape[1]
    nclasses = W2.shape[1]
    rows = 1000
    grid = n // rows
    return pl.pallas_call(
        _mlp_body,
        grid=(grid,),
        in_specs=[
            pl.BlockSpec((rows, d_feat), lambda i: (i, 0)),
            pl.BlockSpec((d_feat, hidden), lambda i: (0, 0)),
            pl.BlockSpec((hidden, nclasses), lambda i: (0, 0)),
        ],
        out_specs=pl.BlockSpec((rows, nclasses), lambda i: (i, 0)),
        out_shape=jax.ShapeDtypeStruct((n, nclasses), jnp.float32),
    )(x, W1, W2)


def _rsqrt_newton(d):
    i = plsc.bitcast(d, jnp.int32)
    i = 0x5F3759DF - lax.shift_right_logical(i, 1)
    y = plsc.bitcast(i, jnp.float32)
    for _ in range(3):
        y = y * (1.5 - 0.5 * d * y * y)
    return y


def _propagate(h_hbm, src_hbm, dst_hbm, n, nchunk):
    rows_pt = n // NTILE
    nclasses = h_hbm.shape[1]
    tab_rows = n + PAD_ROWS

    mesh = plsc.VectorSubcoreMesh(
        core_axis_name="c", subcore_axis_name="s",
        num_cores=NSC, num_subcores=NTILE)

    @functools.partial(
        pl.kernel,
        mesh=mesh,
        compiler_params=pltpu.CompilerParams(
            needs_layout_passes=False, use_tc_tiling_on_sc=False),
        out_type=jax.ShapeDtypeStruct((NTILE, rows_pt, nclasses), jnp.float32),
        scratch_types=[
            pltpu.VMEM_SHARED((tab_rows, nclasses), jnp.float32),
            pltpu.VMEM_SHARED((tab_rows, nclasses), jnp.float32),
            pltpu.VMEM((nchunk, CHUNK), jnp.int32),
            pltpu.VMEM((nchunk, CHUNK), jnp.int32),
            pltpu.VMEM((CHUNK, nclasses), jnp.float32),
            pltpu.VMEM((CHUNK, nclasses), jnp.float32),
            pltpu.VMEM((CHUNK, nclasses), jnp.float32),
            pltpu.VMEM((n // NTILE, nclasses), jnp.float32),
            pltpu.VMEM((n // NTILE, nclasses), jnp.float32),
            pltpu.VMEM((n // NTILE, nclasses), jnp.float32),
            pltpu.VMEM((n // NTILE, nclasses), jnp.float32),
            pltpu.VMEM((n // NTILE, nclasses), jnp.float32),
            pltpu.SemaphoreType.DMA,
            pltpu.SemaphoreType.DMA,
        ],
    )
    def k(h_r, src_r, dst_r, out_r,
          y_tab, s_tab, src_v, dst_v, rows_a, rows_b, ones_v, zeros_v,
          h_v, dx_v, sb_v, yb_v, sem_a, sem_b):
        wid = lax.axis_index("s")
        cid = lax.axis_index("c")
        r0 = wid * rows_pt

        pltpu.sync_copy(src_r.at[wid], src_v)
        pltpu.sync_copy(dst_r.at[wid], dst_v)
        pltpu.sync_copy(h_r.at[wid], h_v)

        def fill_ones(i, _):
            ones_v[i, :] = jnp.full((LANES,), 1.0, jnp.float32)
            return 0
        lax.fori_loop(0, CHUNK, fill_ones, 0)

        def fill_zeros(i, _):
            zeros_v[i, :] = jnp.zeros((LANES,), jnp.float32)
            return 0
        lax.fori_loop(0, rows_pt, fill_zeros, 0)

        pltpu.sync_copy(zeros_v, s_tab.at[pl.ds(r0, rows_pt)])
        plsc.subcore_barrier()

        def deg_step(c, _):
            pltpu.sync_copy(ones_v, s_tab.at[dst_v.at[c]], add=True)
            return 0
        lax.fori_loop(0, nchunk, deg_step, 0)
        plsc.subcore_barrier()

        pltpu.sync_copy(s_tab.at[pl.ds(r0, rows_pt)], sb_v)
        pltpu.sync_copy(zeros_v, s_tab.at[pl.ds(r0, rows_pt)])

        def init_step(f, _):
            deg = sb_v[f, :] + 1.0
            dx = _rsqrt_newton(deg)
            dx_v[f, :] = dx
            sb_v[f, :] = dx * h_v[f, :]
            return 0
        lax.fori_loop(0, rows_pt, init_step, 0)
        pltpu.sync_copy(sb_v, y_tab.at[pl.ds(r0, rows_pt)])
        plsc.subcore_barrier()

        for _ in range(NITER):
            pltpu.async_copy(y_tab.at[src_v.at[0]], rows_a, sem_a)

            def edge_pair(i, _):
                c = 2 * i
                pltpu.async_copy(y_tab.at[src_v.at[c + 1]], rows_b, sem_b)
                pltpu.make_async_copy(y_tab.at[src_v.at[c]], rows_a, sem_a).wait()
                pltpu.sync_copy(rows_a, s_tab.at[dst_v.at[c]], add=True)

                @pl.when(c + 2 < nchunk)
                def _():
                    pltpu.async_copy(y_tab.at[src_v.at[c + 2]], rows_a, sem_a)

                pltpu.make_async_copy(y_tab.at[src_v.at[c + 1]], rows_b, sem_b).wait()
                pltpu.sync_copy(rows_b, s_tab.at[dst_v.at[c + 1]], add=True)
                return 0
            lax.fori_loop(0, nchunk // 2, edge_pair, 0)
            plsc.subcore_barrier()

            pltpu.sync_copy(s_tab.at[pl.ds(r0, rows_pt)], sb_v)
            pltpu.sync_copy(y_tab.at[pl.ds(r0, rows_pt)], yb_v)
            pltpu.sync_copy(zeros_v, s_tab.at[pl.ds(r0, rows_pt)])

            def upd_step(f, _):
                dx = dx_v[f, :]
                v = sb_v[f, :] + yb_v[f, :]
                sb_v[f, :] = ((1.0 - ALPHA) * dx * dx) * v + (ALPHA * dx) * h_v[f, :]
                return 0
            lax.fori_loop(0, rows_pt, upd_step, 0)
            pltpu.sync_copy(sb_v, y_tab.at[pl.ds(r0, rows_pt)])
            plsc.subcore_barrier()

        def out_step(f, _):
            yb_v[f, :] = sb_v[f, :] / dx_v[f, :]
            return 0
        lax.fori_loop(0, rows_pt, out_step, 0)

        @pl.when(cid == 0)
        def _():
            pltpu.sync_copy(yb_v, out_r.at[wid])

    out3 = k(h_hbm.reshape(NTILE, rows_pt, nclasses), src_hbm, dst_hbm)
    return out3.reshape(n, nclasses)


def kernel(x, edge_index, W1, W2):
    n = x.shape[0]
    e = edge_index.shape[1]
    assert n % NTILE == 0

    h = _mlp(x, W1, W2)

    per_tile = -(-e // (NTILE * 2 * CHUNK)) * (2 * CHUNK)
    e_pad = per_tile * NTILE
    pad = jnp.arange(e_pad - e, dtype=jnp.int32) % PAD_ROWS + n
    src = jnp.concatenate([edge_index[0], pad]).reshape(NTILE, per_tile // CHUNK, CHUNK)
    dst = jnp.concatenate([edge_index[1], pad]).reshape(NTILE, per_tile // CHUNK, CHUNK)

    return _propagate(h, src, dst, n, per_tile // CHUNK)
